# baseline (speedup 1.0000x reference)
import functools

import jax
import jax.numpy as jnp
from jax import lax
from jax.experimental import pallas as pl
from jax.experimental.pallas import tpu as pltpu
from jax.experimental.pallas import tpu_sc as plsc

N = 50000
D = 128
E = 200000
R = 3

NC = 2
NS = 16
NPAD = 51200
TROWS = NPAD // NS
ZR = 400
EPT = 12544
NCHUNK = EPT // 128
EP = EPT * NS
PAD_NODE = 50100
KB = 32
NKB = D // KB
BNX = 400

_mesh = plsc.VectorSubcoreMesh(core_axis_name="c", subcore_axis_name="s")


@functools.partial(
    pl.kernel,
    out_type=jax.ShapeDtypeStruct((2 * R * NPAD,), jnp.float32),
    mesh=_mesh,
    scratch_types=[
        pltpu.VMEM_SHARED((NPAD,), jnp.float32),
        pltpu.VMEM_SHARED((NPAD,), jnp.float32),
        pltpu.VMEM_SHARED((NPAD,), jnp.float32),
        pltpu.VMEM((NCHUNK, 128), jnp.int32),
        pltpu.VMEM((TROWS,), jnp.float32),
        pltpu.VMEM((128,), jnp.float32),
        pltpu.SemaphoreType.DMA,
    ],
)
def _sc_hist(idx_all, deg, h0, h1, h2, idxv, zrow, ones, sem):
    c = lax.axis_index("c")
    s = lax.axis_index("s")
    hs = [h0, h1, h2]

    @pl.loop(0, TROWS // 16)
    def _(i):
        zrow[pl.ds(i * 16, 16)] = jnp.zeros((16,), jnp.float32)

    @pl.loop(0, 8)
    def _(i):
        ones[pl.ds(i * 16, 16)] = jnp.ones((16,), jnp.float32)

    for a in range(3):
        off = pl.multiple_of(s * TROWS, 128)
        pltpu.sync_copy(zrow, hs[a].at[pl.ds(off, TROWS)])
    plsc.subcore_barrier()

    for a in range(3):
        g = 3 * c + a
        pltpu.sync_copy(idx_all.at[g, s], idxv)

        @pl.loop(0, NCHUNK)
        def _(j):
            pltpu.async_copy(ones, hs[a].at[idxv.at[j]], sem, add=True)

        @pl.loop(0, NCHUNK)
        def _(j):
            pltpu.make_async_copy(ones, hs[a].at[idxv.at[j]], sem).wait()

    plsc.subcore_barrier()
    for a in range(3):
        g = 3 * c + a
        src_off = pl.multiple_of(s * TROWS, 128)
        dst_off = pl.multiple_of(g * NPAD + s * TROWS, 128)
        pltpu.sync_copy(hs[a].at[pl.ds(src_off, TROWS)],
                        deg.at[pl.ds(dst_off, TROWS)])


def _xn_body(x_ref, dego_ref, xn0_ref, xn1_ref, xn2_ref):
    xv = x_ref[...]
    d = dego_ref[:, 0, 0, :]
    outs = [xn0_ref, xn1_ref, xn2_ref]
    for r in range(R):
        dr = d[r]
        nrm = jnp.where(dr > 0.0, lax.rsqrt(jnp.maximum(dr, 1.0)), 0.0)
        outs[r][...] = xv * nrm[:, None]


def _run_xn(x, dego_st):
    nb = N // BNX
    bn = BNX
    shp = jax.ShapeDtypeStruct((N, D), jnp.float32)
    return pl.pallas_call(
        _xn_body,
        grid=(nb,),
        in_specs=[
            pl.BlockSpec((bn, D), lambda i: (i, 0)),
            pl.BlockSpec((R, 1, 1, bn), lambda i: (0, i, 0, 0)),
        ],
        out_specs=[pl.BlockSpec((bn, D), lambda i: (i, 0))] * R,
        out_shape=[shp, shp, shp],
    )(x, dego_st)


def _agg_body(xn0, xn1, xn2, e0, e1, e2, a0, a1, a2,
              acc, icb, rows, zbuf, isems, gsems, ssems):
    c = lax.axis_index("c")
    s = lax.axis_index("s")
    xns = [xn0, xn1, xn2]
    epks = [e0, e1, e2]
    outs = [a0, a1, a2]
    z16 = jnp.zeros((16,), jnp.float32)

    @pl.loop(0, ZR)
    def _(i):
        zbuf[i, pl.ds(0, 16)] = z16
        zbuf[i, pl.ds(16, 16)] = z16

    for r in range(R):
        epk = epks[r]
        xn = xns[r]
        for p in range(2):
            kk = 2 * c + p

            @pl.loop(0, TROWS // ZR)
            def _(q):
                pltpu.sync_copy(zbuf, acc.at[pl.ds(s * TROWS + q * ZR, ZR)])

            plsc.subcore_barrier()

            def fetch(j, b6):
                pltpu.async_copy(epk.at[kk, s, j], icb.at[b6],
                                 isems.at[b6])

            def visit(j, u):
                b3 = u % 3
                b6 = u % 6

                @pl.when(jnp.logical_and(j >= 3, j < NCHUNK + 3))
                def _():
                    pltpu.make_async_copy(
                        rows.at[b3], acc.at[icb.at[b6, pl.ds(128, 128)]],
                        ssems.at[b3]).wait()

                @pl.when(j < NCHUNK)
                def _():
                    pltpu.make_async_copy(epk.at[kk, s, 0], icb.at[b6],
                                          isems.at[b6]).wait()
                    pltpu.async_copy(xn.at[icb.at[b6, pl.ds(0, 128)]],
                                     rows.at[b3], gsems.at[b3])

                bp3 = (u + 2) % 3
                bp6 = (u + 5) % 6

                @pl.when(jnp.logical_and(j >= 1, j < NCHUNK + 1))
                def _():
                    pltpu.make_async_copy(xn.at[icb.at[bp6, pl.ds(0, 128)]],
                                          rows.at[bp3], gsems.at[bp3]).wait()
                    pltpu.async_copy(rows.at[bp3],
                                     acc.at[icb.at[bp6, pl.ds(128, 128)]],
                                     ssems.at[bp3], add=True)

                @pl.when(j + 2 < NCHUNK)
                def _():
                    fetch(j + 2, (u + 2) % 6)

            fetch(0, 0)
            fetch(1, 1)

            @pl.loop(0, (NCHUNK + 4 + 5) // 6)
            def _(i):
                for u in range(6):
                    visit(6 * i + u, u)

            plsc.subcore_barrier()
            pltpu.sync_copy(
                acc.at[pl.ds(s * TROWS, TROWS)],
                outs[r].at[pl.ds(s * TROWS, TROWS), pl.ds(kk * KB, KB)])


def _run_agg(xns, epks):
    shp = jax.ShapeDtypeStruct((NPAD, D), jnp.float32)
    k = pl.kernel(
        _agg_body,
        out_type=(shp, shp, shp),
        mesh=_mesh,
        compiler_params=pltpu.CompilerParams(use_tc_tiling_on_sc=False),
        scratch_types=[
            pltpu.VMEM_SHARED((NPAD, KB), jnp.float32),
            pltpu.VMEM((6, 256), jnp.int32),
            pltpu.VMEM((3, 128, KB), jnp.float32),
            pltpu.VMEM((ZR, KB), jnp.float32),
            pltpu.SemaphoreType.DMA((6,)),
            pltpu.SemaphoreType.DMA((3,)),
            pltpu.SemaphoreType.DMA((3,)),
        ],
    )
    return k(*xns, *epks)


def _out_body(a0_ref, a1_ref, a2_ref, w_ref, degi_ref, bsum_ref, out_ref):
    d = degi_ref[:, 0, 0, :]
    bn = out_ref.shape[0]
    acc = jnp.zeros((bn, D), jnp.float32)
    ars = [a0_ref, a1_ref, a2_ref]
    for r in range(R):
        dr = d[r]
        nrm = jnp.where(dr > 0.0, lax.rsqrt(jnp.maximum(dr, 1.0)), 0.0)
        t = jnp.dot(ars[r][...], w_ref[r],
                    preferred_element_type=jnp.float32)
        acc = acc + t * nrm[:, None]
    out_ref[...] = acc + bsum_ref[...]


def _run_out(a_list, w_all, degi_st, bsum):
    nb = N // BNX
    bn = BNX
    return pl.pallas_call(
        _out_body,
        grid=(nb,),
        in_specs=[
            pl.BlockSpec((bn, D), lambda i: (i, 0)),
            pl.BlockSpec((bn, D), lambda i: (i, 0)),
            pl.BlockSpec((bn, D), lambda i: (i, 0)),
            pl.BlockSpec((R, D, D), lambda i: (0, 0, 0)),
            pl.BlockSpec((R, 1, 1, bn), lambda i: (0, i, 0, 0)),
            pl.BlockSpec((1, D), lambda i: (0, 0)),
        ],
        out_specs=pl.BlockSpec((bn, D), lambda i: (i, 0)),
        out_shape=jax.ShapeDtypeStruct((N, D), jnp.float32),
    )(*a_list, w_all, degi_st, bsum)


def kernel(x, edge_index_r0, edge_index_r1, edge_index_r2,
           W_r0, W_r1, W_r2, b_r0, b_r1, b_r2):
    eis = [edge_index_r0, edge_index_r1, edge_index_r2]
    pes, epks = [], []
    koff = jnp.arange(NKB, dtype=jnp.int32)[:, None, None, None]
    padblk = jnp.broadcast_to(
        jnp.array([[0], [PAD_NODE]], jnp.int32), (2, EP - E))
    padhist = jnp.full((2, EP - E), PAD_NODE, jnp.int32)
    for ei in eis:
        pe = jnp.concatenate([ei, padblk], axis=1).reshape(
            2, NS, NCHUNK, 128)
        pes.append(jnp.concatenate([ei, padhist], axis=1).reshape(
            2, NS, NCHUNK, 128))
        epks.append(jnp.concatenate(
            [pe[0:1] * 4 + koff,
             jnp.broadcast_to(pe[1:2], (NKB, NS, NCHUNK, 128))], axis=-1))
    idx_all = jnp.stack(pes, axis=0).reshape(2 * R, NS, NCHUNK, 128)

    deg = _sc_hist(idx_all).reshape(2 * R, NPAD)

    nbx = N // BNX
    dego_st = deg[0::2, :N].reshape(R, nbx, 1, BNX)
    degi_st = deg[1::2, :N].reshape(R, nbx, 1, BNX)

    xn_list = _run_xn(x, dego_st)
    xn3 = [xn.reshape(NKB * N, KB) for xn in xn_list]

    a_list = _run_agg(xn3, epks)

    w_all = jnp.stack([W_r0, W_r1, W_r2], axis=0)
    bsum = (b_r0 + b_r1 + b_r2).reshape(1, D)
    return _run_out(a_list, w_all, degi_st, bsum)


# speedup vs baseline: 1.1612x; 1.1612x over previous
import functools

import jax
import jax.numpy as jnp
from jax import lax
from jax.experimental import pallas as pl
from jax.experimental.pallas import tpu as pltpu
from jax.experimental.pallas import tpu_sc as plsc

N = 50000
D = 128
E = 200000
R = 3

NC = 2
NS = 16
NPAD = 51200
TROWS = NPAD // NS
ZR = 400
EPT = 12544
NCHUNK = EPT // 128
EP = EPT * NS
PAD_NODE = 50100
KB = 32
NKB = D // KB
BNX = 2000

_mesh = plsc.VectorSubcoreMesh(core_axis_name="c", subcore_axis_name="s")


@functools.partial(
    pl.kernel,
    out_type=jax.ShapeDtypeStruct((2 * R * NPAD,), jnp.float32),
    mesh=_mesh,
    scratch_types=[
        pltpu.VMEM_SHARED((NPAD,), jnp.float32),
        pltpu.VMEM_SHARED((NPAD,), jnp.float32),
        pltpu.VMEM_SHARED((NPAD,), jnp.float32),
        pltpu.VMEM((NCHUNK, 128), jnp.int32),
        pltpu.VMEM((TROWS,), jnp.float32),
        pltpu.VMEM((128,), jnp.float32),
        pltpu.SemaphoreType.DMA,
    ],
)
def _sc_hist(idx_all, deg, h0, h1, h2, idxv, zrow, ones, sem):
    c = lax.axis_index("c")
    s = lax.axis_index("s")
    hs = [h0, h1, h2]

    @pl.loop(0, TROWS // 16)
    def _(i):
        zrow[pl.ds(i * 16, 16)] = jnp.zeros((16,), jnp.float32)

    @pl.loop(0, 8)
    def _(i):
        ones[pl.ds(i * 16, 16)] = jnp.ones((16,), jnp.float32)

    for a in range(3):
        off = pl.multiple_of(s * TROWS, 128)
        pltpu.sync_copy(zrow, hs[a].at[pl.ds(off, TROWS)])
    plsc.subcore_barrier()

    for a in range(3):
        g = 3 * c + a
        pltpu.sync_copy(idx_all.at[g, s], idxv)

        @pl.loop(0, NCHUNK)
        def _(j):
            pltpu.async_copy(ones, hs[a].at[idxv.at[j]], sem, add=True)

        @pl.loop(0, NCHUNK)
        def _(j):
            pltpu.make_async_copy(ones, hs[a].at[idxv.at[j]], sem).wait()

    plsc.subcore_barrier()
    for a in range(3):
        g = 3 * c + a
        src_off = pl.multiple_of(s * TROWS, 128)
        dst_off = pl.multiple_of(g * NPAD + s * TROWS, 128)
        pltpu.sync_copy(hs[a].at[pl.ds(src_off, TROWS)],
                        deg.at[pl.ds(dst_off, TROWS)])


def _xn_body(x_ref, dego_ref, xn0_ref, xn1_ref, xn2_ref):
    xv = x_ref[...]
    d = dego_ref[:, 0, 0, :]
    outs = [xn0_ref, xn1_ref, xn2_ref]
    for r in range(R):
        dr = d[r]
        nrm = jnp.where(dr > 0.0, lax.rsqrt(jnp.maximum(dr, 1.0)), 0.0)
        outs[r][...] = xv * nrm[:, None]


def _run_xn(x, dego_st):
    nb = N // BNX
    bn = BNX
    shp = jax.ShapeDtypeStruct((N, D), jnp.float32)
    return pl.pallas_call(
        _xn_body,
        grid=(nb,),
        in_specs=[
            pl.BlockSpec((bn, D), lambda i: (i, 0)),
            pl.BlockSpec((R, 1, 1, bn), lambda i: (0, i, 0, 0)),
        ],
        out_specs=[pl.BlockSpec((bn, D), lambda i: (i, 0))] * R,
        out_shape=[shp, shp, shp],
    )(x, dego_st)


def _agg_body(xn0, xn1, xn2, e0, e1, e2, a0, a1, a2,
              acc, icb, rows, zbuf, isems, gsems, ssems):
    c = lax.axis_index("c")
    s = lax.axis_index("s")
    xns = [xn0, xn1, xn2]
    epks = [e0, e1, e2]
    outs = [a0, a1, a2]
    z16 = jnp.zeros((16,), jnp.float32)

    @pl.loop(0, ZR)
    def _(i):
        zbuf[i, pl.ds(0, 16)] = z16
        zbuf[i, pl.ds(16, 16)] = z16

    for r in range(R):
        epk = epks[r]
        xn = xns[r]
        for p in range(2):
            kk = 2 * c + p

            @pl.loop(0, TROWS // ZR)
            def _(q):
                pltpu.sync_copy(zbuf, acc.at[pl.ds(s * TROWS + q * ZR, ZR)])

            plsc.subcore_barrier()

            def fetch(j, b6):
                pltpu.async_copy(epk.at[kk, s, j], icb.at[b6],
                                 isems.at[b6])

            def visit(j, u):
                b3 = u % 3
                b6 = u % 6

                @pl.when(jnp.logical_and(j >= 3, j < NCHUNK + 3))
                def _():
                    pltpu.make_async_copy(
                        rows.at[b3], acc.at[icb.at[b6, pl.ds(128, 128)]],
                        ssems.at[b3]).wait()

                @pl.when(j < NCHUNK)
                def _():
                    pltpu.make_async_copy(epk.at[kk, s, 0], icb.at[b6],
                                          isems.at[b6]).wait()
                    pltpu.async_copy(xn.at[icb.at[b6, pl.ds(0, 128)]],
                                     rows.at[b3], gsems.at[b3])

                bp3 = (u + 2) % 3
                bp6 = (u + 5) % 6

                @pl.when(jnp.logical_and(j >= 1, j < NCHUNK + 1))
                def _():
                    pltpu.make_async_copy(xn.at[icb.at[bp6, pl.ds(0, 128)]],
                                          rows.at[bp3], gsems.at[bp3]).wait()
                    pltpu.async_copy(rows.at[bp3],
                                     acc.at[icb.at[bp6, pl.ds(128, 128)]],
                                     ssems.at[bp3], add=True)

                @pl.when(j + 2 < NCHUNK)
                def _():
                    fetch(j + 2, (u + 2) % 6)

            fetch(0, 0)
            fetch(1, 1)

            @pl.loop(0, (NCHUNK + 4 + 5) // 6)
            def _(i):
                for u in range(6):
                    visit(6 * i + u, u)

            plsc.subcore_barrier()
            pltpu.sync_copy(
                acc.at[pl.ds(s * TROWS, TROWS)],
                outs[r].at[pl.ds(s * TROWS, TROWS), pl.ds(kk * KB, KB)])


def _run_agg(xns, epks):
    shp = jax.ShapeDtypeStruct((NPAD, D), jnp.float32)
    k = pl.kernel(
        _agg_body,
        out_type=(shp, shp, shp),
        mesh=_mesh,
        compiler_params=pltpu.CompilerParams(use_tc_tiling_on_sc=False),
        scratch_types=[
            pltpu.VMEM_SHARED((NPAD, KB), jnp.float32),
            pltpu.VMEM((6, 256), jnp.int32),
            pltpu.VMEM((3, 128, KB), jnp.float32),
            pltpu.VMEM((ZR, KB), jnp.float32),
            pltpu.SemaphoreType.DMA((6,)),
            pltpu.SemaphoreType.DMA((3,)),
            pltpu.SemaphoreType.DMA((3,)),
        ],
    )
    return k(*xns, *epks)


def _out_body(a0_ref, a1_ref, a2_ref, w_ref, degi_ref, bsum_ref, out_ref):
    d = degi_ref[:, 0, 0, :]
    bn = out_ref.shape[0]
    acc = jnp.zeros((bn, D), jnp.float32)
    ars = [a0_ref, a1_ref, a2_ref]
    for r in range(R):
        dr = d[r]
        nrm = jnp.where(dr > 0.0, lax.rsqrt(jnp.maximum(dr, 1.0)), 0.0)
        t = jnp.dot(ars[r][...], w_ref[r],
                    preferred_element_type=jnp.float32)
        acc = acc + t * nrm[:, None]
    out_ref[...] = acc + bsum_ref[...]


def _run_out(a_list, w_all, degi_st, bsum):
    nb = N // BNX
    bn = BNX
    return pl.pallas_call(
        _out_body,
        grid=(nb,),
        in_specs=[
            pl.BlockSpec((bn, D), lambda i: (i, 0)),
            pl.BlockSpec((bn, D), lambda i: (i, 0)),
            pl.BlockSpec((bn, D), lambda i: (i, 0)),
            pl.BlockSpec((R, D, D), lambda i: (0, 0, 0)),
            pl.BlockSpec((R, 1, 1, bn), lambda i: (0, i, 0, 0)),
            pl.BlockSpec((1, D), lambda i: (0, 0)),
        ],
        out_specs=pl.BlockSpec((bn, D), lambda i: (i, 0)),
        out_shape=jax.ShapeDtypeStruct((N, D), jnp.float32),
    )(*a_list, w_all, degi_st, bsum)


def kernel(x, edge_index_r0, edge_index_r1, edge_index_r2,
           W_r0, W_r1, W_r2, b_r0, b_r1, b_r2):
    eis = [edge_index_r0, edge_index_r1, edge_index_r2]
    pes, epks = [], []
    koff = jnp.arange(NKB, dtype=jnp.int32)[:, None, None, None]
    padblk = jnp.broadcast_to(
        jnp.array([[0], [PAD_NODE]], jnp.int32), (2, EP - E))
    padhist = jnp.full((2, EP - E), PAD_NODE, jnp.int32)
    for ei in eis:
        pe = jnp.concatenate([ei, padblk], axis=1).reshape(
            2, NS, NCHUNK, 128)
        pes.append(jnp.concatenate([ei, padhist], axis=1).reshape(
            2, NS, NCHUNK, 128))
        epks.append(jnp.concatenate(
            [pe[0:1] * 4 + koff,
             jnp.broadcast_to(pe[1:2], (NKB, NS, NCHUNK, 128))], axis=-1))
    idx_all = jnp.stack(pes, axis=0).reshape(2 * R, NS, NCHUNK, 128)

    deg = _sc_hist(idx_all).reshape(2 * R, NPAD)

    nbx = N // BNX
    dego_st = deg[0::2, :N].reshape(R, nbx, 1, BNX)
    degi_st = deg[1::2, :N].reshape(R, nbx, 1, BNX)

    xn_list = _run_xn(x, dego_st)
    xn3 = [xn.reshape(NKB * N, KB) for xn in xn_list]

    a_list = _run_agg(xn3, epks)

    w_all = jnp.stack([W_r0, W_r1, W_r2], axis=0)
    bsum = (b_r0 + b_r1 + b_r2).reshape(1, D)
    return _run_out(a_list, w_all, degi_st, bsum)


# speedup vs baseline: 1.3125x; 1.1303x over previous
import functools

import jax
import jax.numpy as jnp
from jax import lax
from jax.experimental import pallas as pl
from jax.experimental.pallas import tpu as pltpu
from jax.experimental.pallas import tpu_sc as plsc

N = 50000
D = 128
E = 200000
R = 3

NC = 2
NS = 16
NPAD = 51200
TROWS = NPAD // NS
ZR = 200
EPT = 12544
NCHUNK = EPT // 128
EP = EPT * NS
PAD_NODE = 50100
KB = 32
NKB = D // KB
BNX = 2000

_mesh = plsc.VectorSubcoreMesh(core_axis_name="c", subcore_axis_name="s")


@functools.partial(
    pl.kernel,
    out_type=jax.ShapeDtypeStruct((2 * R * NPAD,), jnp.float32),
    mesh=_mesh,
    scratch_types=[
        pltpu.VMEM_SHARED((NPAD,), jnp.float32),
        pltpu.VMEM_SHARED((NPAD,), jnp.float32),
        pltpu.VMEM_SHARED((NPAD,), jnp.float32),
        pltpu.VMEM((NCHUNK, 128), jnp.int32),
        pltpu.VMEM((TROWS,), jnp.float32),
        pltpu.VMEM((128,), jnp.float32),
        pltpu.SemaphoreType.DMA,
    ],
)
def _sc_hist(idx_all, deg, h0, h1, h2, idxv, zrow, ones, sem):
    c = lax.axis_index("c")
    s = lax.axis_index("s")
    hs = [h0, h1, h2]

    @pl.loop(0, TROWS // 16)
    def _(i):
        zrow[pl.ds(i * 16, 16)] = jnp.zeros((16,), jnp.float32)

    @pl.loop(0, 8)
    def _(i):
        ones[pl.ds(i * 16, 16)] = jnp.ones((16,), jnp.float32)

    for a in range(3):
        off = pl.multiple_of(s * TROWS, 128)
        pltpu.sync_copy(zrow, hs[a].at[pl.ds(off, TROWS)])
    plsc.subcore_barrier()

    for a in range(3):
        g = 3 * c + a
        pltpu.sync_copy(idx_all.at[g, s], idxv)

        @pl.loop(0, NCHUNK)
        def _(j):
            pltpu.async_copy(ones, hs[a].at[idxv.at[j]], sem, add=True)

        @pl.loop(0, NCHUNK)
        def _(j):
            pltpu.make_async_copy(ones, hs[a].at[idxv.at[j]], sem).wait()

    plsc.subcore_barrier()
    for a in range(3):
        g = 3 * c + a
        src_off = pl.multiple_of(s * TROWS, 128)
        dst_off = pl.multiple_of(g * NPAD + s * TROWS, 128)
        pltpu.sync_copy(hs[a].at[pl.ds(src_off, TROWS)],
                        deg.at[pl.ds(dst_off, TROWS)])


def _xn_body(x_ref, dego_ref, xn0_ref, xn1_ref, xn2_ref):
    xv = x_ref[...]
    d = dego_ref[:, 0, 0, :]
    outs = [xn0_ref, xn1_ref, xn2_ref]
    for r in range(R):
        dr = d[r]
        nrm = jnp.where(dr > 0.0, lax.rsqrt(jnp.maximum(dr, 1.0)), 0.0)
        outs[r][...] = xv * nrm[:, None]


def _run_xn(x, dego_st):
    nb = N // BNX
    bn = BNX
    shp = jax.ShapeDtypeStruct((N, D), jnp.float32)
    return pl.pallas_call(
        _xn_body,
        grid=(nb,),
        in_specs=[
            pl.BlockSpec((bn, D), lambda i: (i, 0)),
            pl.BlockSpec((R, 1, 1, bn), lambda i: (0, i, 0, 0)),
        ],
        out_specs=[pl.BlockSpec((bn, D), lambda i: (i, 0))] * R,
        out_shape=[shp, shp, shp],
    )(x, dego_st)


def _agg_body(xn0, xn1, xn2, e0, e1, e2, a0, a1, a2,
              acc, icb, rows, zbuf, isems, gsems, ssems):
    c = lax.axis_index("c")
    s = lax.axis_index("s")
    xns = [xn0, xn1, xn2]
    epks = [e0, e1, e2]
    outs = [a0, a1, a2]
    z16 = jnp.zeros((16,), jnp.float32)

    @pl.loop(0, ZR)
    def _(i):
        zbuf[i, pl.ds(0, 16)] = z16
        zbuf[i, pl.ds(16, 16)] = z16

    for r in range(R):
        epk = epks[r]
        xn = xns[r]
        for p in range(2):
            kk = 2 * c + p

            @pl.loop(0, TROWS // ZR)
            def _(q):
                pltpu.sync_copy(zbuf, acc.at[pl.ds(s * TROWS + q * ZR, ZR)])

            plsc.subcore_barrier()

            def fetch(j, b8):
                pltpu.async_copy(epk.at[kk, s, j], icb.at[b8],
                                 isems.at[b8])

            def visit(j, u):
                b4 = u % 4
                b8 = u % 8

                @pl.when(jnp.logical_and(j >= 4, j < NCHUNK + 4))
                def _():
                    pltpu.make_async_copy(
                        rows.at[b4], acc.at[icb.at[b8, pl.ds(128, 128)]],
                        ssems.at[b4]).wait()

                @pl.when(j < NCHUNK)
                def _():
                    pltpu.make_async_copy(epk.at[kk, s, 0], icb.at[b8],
                                          isems.at[b8]).wait()
                    pltpu.async_copy(xn.at[icb.at[b8, pl.ds(0, 128)]],
                                     rows.at[b4], gsems.at[b4])

                bs4 = (u + 2) % 4
                bs8 = (u + 6) % 8

                @pl.when(jnp.logical_and(j >= 2, j < NCHUNK + 2))
                def _():
                    pltpu.make_async_copy(xn.at[icb.at[bs8, pl.ds(0, 128)]],
                                          rows.at[bs4], gsems.at[bs4]).wait()
                    pltpu.async_copy(rows.at[bs4],
                                     acc.at[icb.at[bs8, pl.ds(128, 128)]],
                                     ssems.at[bs4], add=True)

                @pl.when(j + 4 < NCHUNK)
                def _():
                    fetch(j + 4, (u + 4) % 8)

            for jj in range(4):
                fetch(jj, jj)

            @pl.loop(0, 13)
            def _(i):
                for u in range(8):
                    visit(8 * i + u, u)

            plsc.subcore_barrier()
            pltpu.sync_copy(
                acc.at[pl.ds(s * TROWS, TROWS)],
                outs[r].at[pl.ds(s * TROWS, TROWS), pl.ds(kk * KB, KB)])


def _run_agg(xns, epks):
    shp = jax.ShapeDtypeStruct((NPAD, D), jnp.float32)
    k = pl.kernel(
        _agg_body,
        out_type=(shp, shp, shp),
        mesh=_mesh,
        compiler_params=pltpu.CompilerParams(use_tc_tiling_on_sc=False),
        scratch_types=[
            pltpu.VMEM_SHARED((NPAD, KB), jnp.float32),
            pltpu.VMEM((8, 256), jnp.int32),
            pltpu.VMEM((4, 128, KB), jnp.float32),
            pltpu.VMEM((ZR, KB), jnp.float32),
            pltpu.SemaphoreType.DMA((8,)),
            pltpu.SemaphoreType.DMA((4,)),
            pltpu.SemaphoreType.DMA((4,)),
        ],
    )
    return k(*xns, *epks)


def _out_body(a0_ref, a1_ref, a2_ref, w_ref, degi_ref, bsum_ref, out_ref):
    d = degi_ref[:, 0, 0, :]
    bn = out_ref.shape[0]
    acc = jnp.zeros((bn, D), jnp.float32)
    ars = [a0_ref, a1_ref, a2_ref]
    for r in range(R):
        dr = d[r]
        nrm = jnp.where(dr > 0.0, lax.rsqrt(jnp.maximum(dr, 1.0)), 0.0)
        t = jnp.dot(ars[r][...], w_ref[r],
                    preferred_element_type=jnp.float32)
        acc = acc + t * nrm[:, None]
    out_ref[...] = acc + bsum_ref[...]


def _run_out(a_list, w_all, degi_st, bsum):
    nb = N // BNX
    bn = BNX
    return pl.pallas_call(
        _out_body,
        grid=(nb,),
        in_specs=[
            pl.BlockSpec((bn, D), lambda i: (i, 0)),
            pl.BlockSpec((bn, D), lambda i: (i, 0)),
            pl.BlockSpec((bn, D), lambda i: (i, 0)),
            pl.BlockSpec((R, D, D), lambda i: (0, 0, 0)),
            pl.BlockSpec((R, 1, 1, bn), lambda i: (0, i, 0, 0)),
            pl.BlockSpec((1, D), lambda i: (0, 0)),
        ],
        out_specs=pl.BlockSpec((bn, D), lambda i: (i, 0)),
        out_shape=jax.ShapeDtypeStruct((N, D), jnp.float32),
    )(*a_list, w_all, degi_st, bsum)


def kernel(x, edge_index_r0, edge_index_r1, edge_index_r2,
           W_r0, W_r1, W_r2, b_r0, b_r1, b_r2):
    eis = [edge_index_r0, edge_index_r1, edge_index_r2]
    pes, epks = [], []
    koff = jnp.arange(NKB, dtype=jnp.int32)[:, None, None, None]
    padblk = jnp.broadcast_to(
        jnp.array([[0], [PAD_NODE]], jnp.int32), (2, EP - E))
    padhist = jnp.full((2, EP - E), PAD_NODE, jnp.int32)
    for ei in eis:
        pe = jnp.concatenate([ei, padblk], axis=1).reshape(
            2, NS, NCHUNK, 128)
        pes.append(jnp.concatenate([ei, padhist], axis=1).reshape(
            2, NS, NCHUNK, 128))
        epks.append(jnp.concatenate(
            [pe[0:1] * 4 + koff,
             jnp.broadcast_to(pe[1:2], (NKB, NS, NCHUNK, 128))], axis=-1))
    idx_all = jnp.stack(pes, axis=0).reshape(2 * R, NS, NCHUNK, 128)

    deg = _sc_hist(idx_all).reshape(2 * R, NPAD)

    nbx = N // BNX
    dego_st = deg[0::2, :N].reshape(R, nbx, 1, BNX)
    degi_st = deg[1::2, :N].reshape(R, nbx, 1, BNX)

    xn_list = _run_xn(x, dego_st)
    xn3 = [xn.reshape(NKB * N, KB) for xn in xn_list]

    a_list = _run_agg(xn3, epks)

    w_all = jnp.stack([W_r0, W_r1, W_r2], axis=0)
    bsum = (b_r0 + b_r1 + b_r2).reshape(1, D)
    return _run_out(a_list, w_all, degi_st, bsum)


# speedup vs baseline: 1.3811x; 1.0523x over previous
import functools

import jax
import jax.numpy as jnp
from jax import lax
from jax.experimental import pallas as pl
from jax.experimental.pallas import tpu as pltpu
from jax.experimental.pallas import tpu_sc as plsc

N = 50000
D = 128
E = 200000
R = 3

NC = 2
NS = 16
NPAD = 51200
TROWS = NPAD // NS
ZR = 100
EPT = 12544
NCHUNK = EPT // 128
EP = EPT * NS
PAD_NODE = 50100
KB = 32
NKB = D // KB
BNX = 2000

_mesh = plsc.VectorSubcoreMesh(core_axis_name="c", subcore_axis_name="s")


@functools.partial(
    pl.kernel,
    out_type=jax.ShapeDtypeStruct((2 * R * NPAD,), jnp.float32),
    mesh=_mesh,
    scratch_types=[
        pltpu.VMEM_SHARED((NPAD,), jnp.float32),
        pltpu.VMEM_SHARED((NPAD,), jnp.float32),
        pltpu.VMEM_SHARED((NPAD,), jnp.float32),
        pltpu.VMEM((NCHUNK, 128), jnp.int32),
        pltpu.VMEM((TROWS,), jnp.float32),
        pltpu.VMEM((128,), jnp.float32),
        pltpu.SemaphoreType.DMA,
    ],
)
def _sc_hist(idx_all, deg, h0, h1, h2, idxv, zrow, ones, sem):
    c = lax.axis_index("c")
    s = lax.axis_index("s")
    hs = [h0, h1, h2]

    @pl.loop(0, TROWS // 16)
    def _(i):
        zrow[pl.ds(i * 16, 16)] = jnp.zeros((16,), jnp.float32)

    @pl.loop(0, 8)
    def _(i):
        ones[pl.ds(i * 16, 16)] = jnp.ones((16,), jnp.float32)

    for a in range(3):
        off = pl.multiple_of(s * TROWS, 128)
        pltpu.sync_copy(zrow, hs[a].at[pl.ds(off, TROWS)])
    plsc.subcore_barrier()

    for a in range(3):
        g = 3 * c + a
        pltpu.sync_copy(idx_all.at[g, s], idxv)

        @pl.loop(0, NCHUNK)
        def _(j):
            pltpu.async_copy(ones, hs[a].at[idxv.at[j]], sem, add=True)

        @pl.loop(0, NCHUNK)
        def _(j):
            pltpu.make_async_copy(ones, hs[a].at[idxv.at[j]], sem).wait()

    plsc.subcore_barrier()
    for a in range(3):
        g = 3 * c + a
        src_off = pl.multiple_of(s * TROWS, 128)
        dst_off = pl.multiple_of(g * NPAD + s * TROWS, 128)
        pltpu.sync_copy(hs[a].at[pl.ds(src_off, TROWS)],
                        deg.at[pl.ds(dst_off, TROWS)])


def _xn_body(x_ref, deg_ref, xn0_ref, xn1_ref, xn2_ref):
    xv = x_ref[...]
    d = deg_ref[:, 0, 0, :]
    outs = [xn0_ref, xn1_ref, xn2_ref]
    for r in range(R):
        dr = d[2 * r]
        nrm = jnp.where(dr > 0.0, lax.rsqrt(jnp.maximum(dr, 1.0)), 0.0)
        outs[r][...] = xv * nrm[:, None]


def _run_xn(x, deg_st):
    nb = N // BNX
    bn = BNX
    shp = jax.ShapeDtypeStruct((N, D), jnp.float32)
    return pl.pallas_call(
        _xn_body,
        grid=(nb,),
        in_specs=[
            pl.BlockSpec((bn, D), lambda i: (i, 0)),
            pl.BlockSpec((2 * R, 1, 1, bn), lambda i: (0, i, 0, 0)),
        ],
        out_specs=[pl.BlockSpec((bn, D), lambda i: (i, 0))] * R,
        out_shape=[shp, shp, shp],
    )(x, deg_st)


def _agg_body(xn0, xn1, xn2, e0, e1, e2, a0, a1, a2,
              acc, icb, rows, zbuf, isems, gsems, ssems):
    c = lax.axis_index("c")
    s = lax.axis_index("s")
    xns = [xn0, xn1, xn2]
    epks = [e0, e1, e2]
    outs = [a0, a1, a2]
    z16 = jnp.zeros((16,), jnp.float32)

    @pl.loop(0, ZR)
    def _(i):
        zbuf[i, pl.ds(0, 16)] = z16
        zbuf[i, pl.ds(16, 16)] = z16

    for r in range(R):
        epk = epks[r]
        xn = xns[r]
        for p in range(2):
            kk = 2 * c + p

            @pl.loop(0, TROWS // ZR)
            def _(q):
                pltpu.sync_copy(zbuf, acc.at[pl.ds(s * TROWS + q * ZR, ZR)])

            plsc.subcore_barrier()

            def fetch(j, b10):
                pltpu.async_copy(epk.at[kk, s, j], icb.at[b10],
                                 isems.at[b10])

            def visit(j, u):
                b5 = u % 5
                b10 = u % 10

                @pl.when(jnp.logical_and(j >= 5, j < NCHUNK + 5))
                def _():
                    pltpu.make_async_copy(
                        rows.at[b5], acc.at[icb.at[b10, pl.ds(128, 128)]],
                        ssems.at[b5]).wait()

                @pl.when(j < NCHUNK)
                def _():
                    pltpu.make_async_copy(epk.at[kk, s, 0], icb.at[b10],
                                          isems.at[b10]).wait()
                    pltpu.async_copy(xn.at[icb.at[b10, pl.ds(0, 128)]],
                                     rows.at[b5], gsems.at[b5])

                bs5 = (u + 2) % 5
                bs10 = (u + 7) % 10

                @pl.when(jnp.logical_and(j >= 3, j < NCHUNK + 3))
                def _():
                    pltpu.make_async_copy(xn.at[icb.at[bs10, pl.ds(0, 128)]],
                                          rows.at[bs5], gsems.at[bs5]).wait()
                    pltpu.async_copy(rows.at[bs5],
                                     acc.at[icb.at[bs10, pl.ds(128, 128)]],
                                     ssems.at[bs5], add=True)

                @pl.when(j + 5 < NCHUNK)
                def _():
                    fetch(j + 5, (u + 5) % 10)

            for jj in range(5):
                fetch(jj, jj)

            @pl.loop(0, 11)
            def _(i):
                for u in range(10):
                    visit(10 * i + u, u)

            plsc.subcore_barrier()
            pltpu.sync_copy(
                acc.at[pl.ds(s * TROWS, TROWS)],
                outs[r].at[pl.ds(s * TROWS, TROWS), pl.ds(kk * KB, KB)])


def _run_agg(xns, epks):
    shp = jax.ShapeDtypeStruct((NPAD, D), jnp.float32)
    k = pl.kernel(
        _agg_body,
        out_type=(shp, shp, shp),
        mesh=_mesh,
        compiler_params=pltpu.CompilerParams(use_tc_tiling_on_sc=False),
        scratch_types=[
            pltpu.VMEM_SHARED((NPAD, KB), jnp.float32),
            pltpu.VMEM((10, 256), jnp.int32),
            pltpu.VMEM((5, 128, KB), jnp.float32),
            pltpu.VMEM((ZR, KB), jnp.float32),
            pltpu.SemaphoreType.DMA((10,)),
            pltpu.SemaphoreType.DMA((5,)),
            pltpu.SemaphoreType.DMA((5,)),
        ],
    )
    return k(*xns, *epks)


def _out_body(a0_ref, a1_ref, a2_ref, w_ref, deg_ref, bsum_ref, out_ref):
    d = deg_ref[:, 0, 0, :]
    bn = out_ref.shape[0]
    acc = jnp.zeros((bn, D), jnp.float32)
    ars = [a0_ref, a1_ref, a2_ref]
    for r in range(R):
        dr = d[2 * r + 1]
        nrm = jnp.where(dr > 0.0, lax.rsqrt(jnp.maximum(dr, 1.0)), 0.0)
        t = jnp.dot(ars[r][...], w_ref[r],
                    preferred_element_type=jnp.float32)
        acc = acc + t * nrm[:, None]
    out_ref[...] = acc + bsum_ref[...]


def _run_out(a_list, w_all, deg_st, bsum):
    nb = N // BNX
    bn = BNX
    return pl.pallas_call(
        _out_body,
        grid=(nb,),
        in_specs=[
            pl.BlockSpec((bn, D), lambda i: (i, 0)),
            pl.BlockSpec((bn, D), lambda i: (i, 0)),
            pl.BlockSpec((bn, D), lambda i: (i, 0)),
            pl.BlockSpec((R, D, D), lambda i: (0, 0, 0)),
            pl.BlockSpec((2 * R, 1, 1, bn), lambda i: (0, i, 0, 0)),
            pl.BlockSpec((1, D), lambda i: (0, 0)),
        ],
        out_specs=pl.BlockSpec((bn, D), lambda i: (i, 0)),
        out_shape=jax.ShapeDtypeStruct((N, D), jnp.float32),
    )(*a_list, w_all, deg_st, bsum)


def kernel(x, edge_index_r0, edge_index_r1, edge_index_r2,
           W_r0, W_r1, W_r2, b_r0, b_r1, b_r2):
    eis = [edge_index_r0, edge_index_r1, edge_index_r2]
    pes, epks = [], []
    koff = jnp.arange(NKB, dtype=jnp.int32)[:, None, None, None]
    padblk = jnp.broadcast_to(
        jnp.array([[0], [PAD_NODE]], jnp.int32), (2, EP - E))
    padhist = jnp.full((2, EP - E), PAD_NODE, jnp.int32)
    for ei in eis:
        pe = jnp.concatenate([ei, padblk], axis=1).reshape(
            2, NS, NCHUNK, 128)
        pes.append(jnp.concatenate([ei, padhist], axis=1).reshape(
            2, NS, NCHUNK, 128))
        epks.append(jnp.concatenate(
            [pe[0:1] * 4 + koff,
             jnp.broadcast_to(pe[1:2], (NKB, NS, NCHUNK, 128))], axis=-1))
    idx_all = jnp.stack(pes, axis=0).reshape(2 * R, NS, NCHUNK, 128)

    deg = _sc_hist(idx_all).reshape(2 * R, NPAD)

    nbx = N // BNX
    deg_st = deg[:, :N].reshape(2 * R, nbx, 1, BNX)

    xn_list = _run_xn(x, deg_st)
    xn3 = [xn.reshape(NKB * N, KB) for xn in xn_list]

    a_list = _run_agg(xn3, epks)

    w_all = jnp.stack([W_r0, W_r1, W_r2], axis=0)
    bsum = (b_r0 + b_r1 + b_r2).reshape(1, D)
    return _run_out(a_list, w_all, deg_st, bsum)
